# baseline (speedup 1.0000x reference)
import jax
import jax.numpy as jnp
from jax.experimental import pallas as pl

DEPTH_ = 1000
ROWS_PER_BLOCK = 256


def _onehot_block(idx_ref, out_ref):
    vals = idx_ref[0, 0, :]
    v2 = vals.reshape(ROWS_PER_BLOCK, 1)
    cols = jax.lax.broadcasted_iota(jnp.int32, (ROWS_PER_BLOCK, DEPTH_), 1)
    hit = (cols == v2) & (v2 != 0)
    out_ref[...] = hit.astype(jnp.float32)


def kernel(inputs):
    b, t = inputs.shape
    n = b * t
    nblocks = n // ROWS_PER_BLOCK
    idx3 = inputs.reshape(nblocks, 1, ROWS_PER_BLOCK)
    out = pl.pallas_call(
        _onehot_block,
        grid=(nblocks,),
        in_specs=[pl.BlockSpec((1, 1, ROWS_PER_BLOCK), lambda i: (i, 0, 0))],
        out_specs=pl.BlockSpec((ROWS_PER_BLOCK, DEPTH_), lambda i: (i, 0)),
        out_shape=jax.ShapeDtypeStruct((n, DEPTH_), jnp.float32),
    )(idx3)
    return out.reshape(b, t, DEPTH_)


# speedup vs baseline: 1.5820x; 1.5820x over previous
import jax
import jax.numpy as jnp
from jax.experimental import pallas as pl

DEPTH_ = 1000
B_BLK = 16


def _onehot_block(idx_ref, out_ref):
    vals = idx_ref[...]
    t = vals.shape[1]
    cols = jax.lax.broadcasted_iota(jnp.int32, (B_BLK, t, DEPTH_), 2)
    v3 = vals[:, :, None]
    hit = (cols == v3) & (v3 != 0)
    out_ref[...] = hit.astype(jnp.float32)


def kernel(inputs):
    b, t = inputs.shape
    out = pl.pallas_call(
        _onehot_block,
        grid=(b // B_BLK,),
        in_specs=[pl.BlockSpec((B_BLK, t), lambda i: (i, 0))],
        out_specs=pl.BlockSpec((B_BLK, t, DEPTH_), lambda i: (i, 0, 0)),
        out_shape=jax.ShapeDtypeStruct((b, t, DEPTH_), jnp.float32),
    )(inputs)
    return out


# speedup vs baseline: 1.6262x; 1.0280x over previous
import jax
import jax.numpy as jnp
from jax.experimental import pallas as pl

DEPTH_ = 1000
B_BLK = 64


def _onehot_block(idx_ref, out_ref):
    vals = idx_ref[...]
    t = vals.shape[1]
    cols = jax.lax.broadcasted_iota(jnp.int32, (B_BLK, t, DEPTH_), 2)
    v3 = vals[:, :, None]
    hit = (cols == v3) & (v3 != 0)
    out_ref[...] = hit.astype(jnp.float32)


def kernel(inputs):
    b, t = inputs.shape
    out = pl.pallas_call(
        _onehot_block,
        grid=(b // B_BLK,),
        in_specs=[pl.BlockSpec((B_BLK, t), lambda i: (i, 0))],
        out_specs=pl.BlockSpec((B_BLK, t, DEPTH_), lambda i: (i, 0, 0)),
        out_shape=jax.ShapeDtypeStruct((b, t, DEPTH_), jnp.float32),
    )(inputs)
    return out
